# baseline (speedup 1.0000x reference)
import jax
import jax.numpy as jnp
from jax import lax
from jax.experimental import pallas as pl
from jax.experimental.pallas import tpu as pltpu
from jax.experimental.pallas import tpu_sc as plsc

N = 32768
D = 768
E = 64
BLOCK_N = 2048

CHUNKS = 4
NC = N // CHUNKS
NW = 32
ROWS_PER_W = NC // NW
GROUPS = ROWS_PER_W // 16


def _matmul_body(x_ref, w_ref, out_ref):
    out_ref[...] = lax.dot_general(
        w_ref[...], x_ref[...], (((1,), (1,)), ((), ())),
        preferred_element_type=jnp.float32)


def _tc_logits_t(x, W):
    return pl.pallas_call(
        _matmul_body,
        grid=(NC // BLOCK_N,),
        in_specs=[
            pl.BlockSpec((BLOCK_N, D), lambda i: (i, 0)),
            pl.BlockSpec((E, D), lambda i: (0, 0)),
        ],
        out_specs=pl.BlockSpec((E, BLOCK_N), lambda i: (0, i)),
        out_shape=jax.ShapeDtypeStruct((E, NC), jnp.float32),
    )(x, W)


def _route_body(lt_hbm, i1_hbm, i2_hbm, g1_hbm, g2_hbm,
                lt_v, i1_v, i2_v, g1_v, g2_v):
    wid = lax.axis_index("s") * 2 + lax.axis_index("c")
    base = wid * ROWS_PER_W
    pltpu.sync_copy(lt_hbm.at[:, pl.ds(base, ROWS_PER_W)], lt_v)

    neg = jnp.full((16,), -jnp.inf, jnp.float32)
    zero = jnp.zeros((16,), jnp.int32)

    def group(g, _):
        row0 = g * 16

        def upd(e, carry):
            m1, a1, m2, a2 = carry
            ev = jnp.full((16,), e, jnp.int32)
            v = lt_v[e, pl.ds(row0, 16)]
            gt1 = v > m1
            gt2 = v > m2
            m2n = jnp.where(gt1, m1, jnp.where(gt2, v, m2))
            a2n = jnp.where(gt1, a1, jnp.where(gt2, ev, a2))
            m1n = jnp.where(gt1, v, m1)
            a1n = jnp.where(gt1, ev, a1)
            return m1n, a1n, m2n, a2n

        m1, a1, m2, a2 = lax.fori_loop(0, E, upd, (neg, zero, neg, zero))
        e2 = jnp.exp(m2 - m1)
        den = 1.0 + e2
        i1_v[pl.ds(row0, 16)] = a1
        i2_v[pl.ds(row0, 16)] = a2
        g1_v[pl.ds(row0, 16)] = 1.0 / den
        g2_v[pl.ds(row0, 16)] = e2 / den
        return 0

    lax.fori_loop(0, GROUPS, group, 0)
    pltpu.sync_copy(i1_v, i1_hbm.at[pl.ds(base, ROWS_PER_W)])
    pltpu.sync_copy(i2_v, i2_hbm.at[pl.ds(base, ROWS_PER_W)])
    pltpu.sync_copy(g1_v, g1_hbm.at[pl.ds(base, ROWS_PER_W)])
    pltpu.sync_copy(g2_v, g2_hbm.at[pl.ds(base, ROWS_PER_W)])


_sc_route = pl.kernel(
    _route_body,
    out_type=[
        jax.ShapeDtypeStruct((NC,), jnp.int32),
        jax.ShapeDtypeStruct((NC,), jnp.int32),
        jax.ShapeDtypeStruct((NC,), jnp.float32),
        jax.ShapeDtypeStruct((NC,), jnp.float32),
    ],
    mesh=plsc.VectorSubcoreMesh(core_axis_name="c", subcore_axis_name="s"),
    scratch_types=[
        pltpu.VMEM((E, ROWS_PER_W), jnp.float32),
        pltpu.VMEM((ROWS_PER_W,), jnp.int32),
        pltpu.VMEM((ROWS_PER_W,), jnp.int32),
        pltpu.VMEM((ROWS_PER_W,), jnp.float32),
        pltpu.VMEM((ROWS_PER_W,), jnp.float32),
    ],
)


@jax.jit
def kernel(x, W):
    parts = []
    for c in range(CHUNKS):
        logits_t = _tc_logits_t(jax.lax.slice_in_dim(x, c * NC, (c + 1) * NC), W)
        parts.append(_sc_route(logits_t))
    i1 = jnp.concatenate([p[0] for p in parts])
    i2 = jnp.concatenate([p[1] for p in parts])
    g1 = jnp.concatenate([p[2] for p in parts])
    g2 = jnp.concatenate([p[3] for p in parts])
    idx = jnp.concatenate([i1[:, None], i2[:, None]], axis=1)
    gates = jnp.concatenate([g1[:, None], g2[:, None]], axis=1)
    return idx, gates


# speedup vs baseline: 1.9833x; 1.9833x over previous
import jax
import jax.numpy as jnp
from jax import lax
from jax.experimental import pallas as pl
from jax.experimental.pallas import tpu as pltpu
from jax.experimental.pallas import tpu_sc as plsc

N = 32768
D = 768
E = 64
BLOCK_N = 2048

NW = 32
ROWS_PER_W = N // NW
GROUPS = ROWS_PER_W // 16


def _matmul_body(x_ref, w_ref, out_ref):
    out_ref[...] = lax.dot_general(
        w_ref[...], x_ref[...], (((1,), (1,)), ((), ())),
        preferred_element_type=jnp.float32)


def _tc_logits_t(x, W):
    return pl.pallas_call(
        _matmul_body,
        grid=(N // BLOCK_N,),
        in_specs=[
            pl.BlockSpec((BLOCK_N, D), lambda i: (i, 0)),
            pl.BlockSpec((E, D), lambda i: (0, 0)),
        ],
        out_specs=pl.BlockSpec((E, BLOCK_N), lambda i: (0, i)),
        out_shape=jax.ShapeDtypeStruct((E, N), jnp.float32),
    )(x, W)


def _merge(A, B):
    m1a, a1a, m2a, a2a = A
    m1b, a1b, m2b, a2b = B
    take = m1a >= m1b
    m1 = jnp.maximum(m1a, m1b)
    a1 = jnp.where(take, a1a, a1b)
    sa = m2a >= m1b
    sb = m1a >= m2b
    m2 = jnp.where(take, jnp.maximum(m2a, m1b), jnp.maximum(m1a, m2b))
    a2 = jnp.where(take, jnp.where(sa, a2a, a1b), jnp.where(sb, a1a, a2b))
    return m1, a1, m2, a2


def _route_body(lt_hbm, i1_hbm, i2_hbm, g1_hbm, g2_hbm,
                lt_v, i1_v, i2_v, g1_v, g2_v):
    wid = lax.axis_index("s") * 2 + lax.axis_index("c")
    base = wid * ROWS_PER_W
    pltpu.sync_copy(lt_hbm.at[:, pl.ds(base, ROWS_PER_W)], lt_v)

    def group(g, _):
        row0 = g * 16

        def pair(j0, j1):
            va = lt_v[j0, pl.ds(row0, 16)]
            vb = lt_v[j1, pl.ds(row0, 16)]
            ge = va >= vb
            ia = jnp.full((16,), j0, jnp.int32)
            ib = jnp.full((16,), j1, jnp.int32)
            return (jnp.maximum(va, vb), jnp.where(ge, ia, ib),
                    jnp.minimum(va, vb), jnp.where(ge, ib, ia))

        def top2_of_8(b):
            t01 = pair(8 * b + 0, 8 * b + 1)
            t23 = pair(8 * b + 2, 8 * b + 3)
            t45 = pair(8 * b + 4, 8 * b + 5)
            t67 = pair(8 * b + 6, 8 * b + 7)
            return _merge(_merge(t01, t23), _merge(t45, t67))

        t = top2_of_8(0)
        for b in range(1, 8):
            t = _merge(t, top2_of_8(b))
        m1, a1, m2, a2 = t

        e2 = jnp.exp(m2 - m1)
        den = 1.0 + e2
        i1_v[pl.ds(row0, 16)] = a1
        i2_v[pl.ds(row0, 16)] = a2
        g1_v[pl.ds(row0, 16)] = 1.0 / den
        g2_v[pl.ds(row0, 16)] = e2 / den
        return 0

    lax.fori_loop(0, GROUPS, group, 0)
    pltpu.sync_copy(i1_v, i1_hbm.at[pl.ds(base, ROWS_PER_W)])
    pltpu.sync_copy(i2_v, i2_hbm.at[pl.ds(base, ROWS_PER_W)])
    pltpu.sync_copy(g1_v, g1_hbm.at[pl.ds(base, ROWS_PER_W)])
    pltpu.sync_copy(g2_v, g2_hbm.at[pl.ds(base, ROWS_PER_W)])


_sc_route = pl.kernel(
    _route_body,
    out_type=[
        jax.ShapeDtypeStruct((N,), jnp.int32),
        jax.ShapeDtypeStruct((N,), jnp.int32),
        jax.ShapeDtypeStruct((N,), jnp.float32),
        jax.ShapeDtypeStruct((N,), jnp.float32),
    ],
    mesh=plsc.VectorSubcoreMesh(core_axis_name="c", subcore_axis_name="s"),
    scratch_types=[
        pltpu.VMEM((E, ROWS_PER_W), jnp.float32),
        pltpu.VMEM((ROWS_PER_W,), jnp.int32),
        pltpu.VMEM((ROWS_PER_W,), jnp.int32),
        pltpu.VMEM((ROWS_PER_W,), jnp.float32),
        pltpu.VMEM((ROWS_PER_W,), jnp.float32),
    ],
)


@jax.jit
def kernel(x, W):
    logits_t = _tc_logits_t(x, W)
    i1, i2, g1, g2 = _sc_route(logits_t)
    idx = jnp.concatenate([i1[:, None], i2[:, None]], axis=1)
    gates = jnp.concatenate([g1[:, None], g2[:, None]], axis=1)
    return idx, gates


# speedup vs baseline: 2.0216x; 1.0193x over previous
import jax
import jax.numpy as jnp
from jax import lax
from jax.experimental import pallas as pl
from jax.experimental.pallas import tpu as pltpu
from jax.experimental.pallas import tpu_sc as plsc

N = 32768
D = 768
E = 64
BLOCK_N = 4096

NW = 32
ROWS_PER_W = N // NW
GROUPS = ROWS_PER_W // 16


def _matmul_body(x_ref, w_ref, out_ref):
    out_ref[...] = lax.dot_general(
        w_ref[...], x_ref[...], (((1,), (1,)), ((), ())),
        preferred_element_type=jnp.float32)


def _tc_logits_t(x, W):
    return pl.pallas_call(
        _matmul_body,
        grid=(N // BLOCK_N,),
        in_specs=[
            pl.BlockSpec((BLOCK_N, D), lambda i: (i, 0)),
            pl.BlockSpec((E, D), lambda i: (0, 0)),
        ],
        out_specs=pl.BlockSpec((E, BLOCK_N), lambda i: (0, i)),
        out_shape=jax.ShapeDtypeStruct((E, N), jnp.float32),
    )(x, W)


def _merge(A, B):
    m1a, a1a, m2a, a2a = A
    m1b, a1b, m2b, a2b = B
    take = m1a >= m1b
    m1 = jnp.maximum(m1a, m1b)
    a1 = jnp.where(take, a1a, a1b)
    sa = m2a >= m1b
    sb = m1a >= m2b
    m2 = jnp.where(take, jnp.maximum(m2a, m1b), jnp.maximum(m1a, m2b))
    a2 = jnp.where(take, jnp.where(sa, a2a, a1b), jnp.where(sb, a1a, a2b))
    return m1, a1, m2, a2


def _route_body(lt_hbm, i1_hbm, i2_hbm, g1_hbm, g2_hbm,
                lt_v, i1_v, i2_v, g1_v, g2_v):
    wid = lax.axis_index("s") * 2 + lax.axis_index("c")
    base = wid * ROWS_PER_W
    pltpu.sync_copy(lt_hbm.at[:, pl.ds(base, ROWS_PER_W)], lt_v)

    def group(g, _):
        row0 = g * 16

        def pair(j0, j1):
            va = lt_v[j0, pl.ds(row0, 16)]
            vb = lt_v[j1, pl.ds(row0, 16)]
            ge = va >= vb
            ia = jnp.full((16,), j0, jnp.int32)
            ib = jnp.full((16,), j1, jnp.int32)
            return (jnp.maximum(va, vb), jnp.where(ge, ia, ib),
                    jnp.minimum(va, vb), jnp.where(ge, ib, ia))

        def top2_of_8(b):
            t01 = pair(8 * b + 0, 8 * b + 1)
            t23 = pair(8 * b + 2, 8 * b + 3)
            t45 = pair(8 * b + 4, 8 * b + 5)
            t67 = pair(8 * b + 6, 8 * b + 7)
            return _merge(_merge(t01, t23), _merge(t45, t67))

        t = top2_of_8(0)
        for b in range(1, 8):
            t = _merge(t, top2_of_8(b))
        m1, a1, m2, a2 = t

        e2 = jnp.exp(m2 - m1)
        den = 1.0 + e2
        i1_v[pl.ds(row0, 16)] = a1
        i2_v[pl.ds(row0, 16)] = a2
        g1_v[pl.ds(row0, 16)] = 1.0 / den
        g2_v[pl.ds(row0, 16)] = e2 / den
        return 0

    lax.fori_loop(0, GROUPS, group, 0)
    pltpu.sync_copy(i1_v, i1_hbm.at[pl.ds(base, ROWS_PER_W)])
    pltpu.sync_copy(i2_v, i2_hbm.at[pl.ds(base, ROWS_PER_W)])
    pltpu.sync_copy(g1_v, g1_hbm.at[pl.ds(base, ROWS_PER_W)])
    pltpu.sync_copy(g2_v, g2_hbm.at[pl.ds(base, ROWS_PER_W)])


_sc_route = pl.kernel(
    _route_body,
    out_type=[
        jax.ShapeDtypeStruct((N,), jnp.int32),
        jax.ShapeDtypeStruct((N,), jnp.int32),
        jax.ShapeDtypeStruct((N,), jnp.float32),
        jax.ShapeDtypeStruct((N,), jnp.float32),
    ],
    mesh=plsc.VectorSubcoreMesh(core_axis_name="c", subcore_axis_name="s"),
    scratch_types=[
        pltpu.VMEM((E, ROWS_PER_W), jnp.float32),
        pltpu.VMEM((ROWS_PER_W,), jnp.int32),
        pltpu.VMEM((ROWS_PER_W,), jnp.int32),
        pltpu.VMEM((ROWS_PER_W,), jnp.float32),
        pltpu.VMEM((ROWS_PER_W,), jnp.float32),
    ],
)


@jax.jit
def kernel(x, W):
    logits_t = _tc_logits_t(x, W)
    i1, i2, g1, g2 = _sc_route(logits_t)
    idx = jnp.concatenate([i1[:, None], i2[:, None]], axis=1)
    gates = jnp.concatenate([g1[:, None], g2[:, None]], axis=1)
    return idx, gates
